# baseline (speedup 1.0000x reference)
import functools

import jax
import jax.numpy as jnp
from jax import lax
from jax.experimental import pallas as pl
from jax.experimental.pallas import tpu as pltpu
from jax.experimental.pallas import tpu_sc as plsc

F32 = jnp.float32
I32 = jnp.int32

N_REAL = 50000
NPAD = 51200
D = 128
H = 128
O = 64
NC = 2
NS = 16
G = 128
TRASH = N_REAL
SLAB = 32
ZR = 64
CRH = 8

EC_ROWS = 2560
EW_ROWS = 1280

MBLK = 512
NBLK = NPAD // MBLK
FPT = NPAD // NS


def _hist_body(csrc, cdst, wsrc, wdst, bsrc, bdst,
               o_cs, o_cd, o_ws, o_wd, o_bs, o_bd,
               h0, h1, h2, h3, ibuf, ones, zbuf):
  core = lax.axis_index("c")
  sid = lax.axis_index("s")
  zh = NPAD // NS

  def fill_ones(i, _):
    ones[pl.ds(i * 16, 16)] = jnp.full((16,), 1.0, F32)
    return 0
  lax.fori_loop(0, G // 16, fill_ones, 0)

  def fill_z(i, _):
    zbuf[pl.ds(i * 16, 16)] = jnp.zeros((16,), F32)
    return 0
  lax.fori_loop(0, zh // 16, fill_z, 0)

  for h in (h0, h1, h2, h3):
    pltpu.sync_copy(zbuf, h.at[pl.ds(sid * zh, zh)])
  plsc.subcore_barrier()

  def accum(idx2d, hist, trows):
    brow = sid * trows

    def chunk(j, _):
      pltpu.sync_copy(idx2d.at[pl.ds(brow + j * CRH, CRH)], ibuf)

      def row(r, _):
        pltpu.sync_copy(ones, hist.at[ibuf.at[r]], add=True)
        return 0
      lax.fori_loop(0, CRH, row, 0)
      return 0
    lax.fori_loop(0, trows // CRH, chunk, 0)

  @pl.when(core == 0)
  def _():
    accum(csrc, h0, EC_ROWS // NS)
    accum(cdst, h1, EC_ROWS // NS)

  @pl.when(core == 1)
  def _():
    accum(wsrc, h0, EW_ROWS // NS)
    accum(wdst, h1, EW_ROWS // NS)
    accum(bsrc, h2, EW_ROWS // NS)
    accum(bdst, h3, EW_ROWS // NS)

  plsc.subcore_barrier()

  def flush(hist, out):
    pltpu.sync_copy(hist.at[pl.ds(sid * zh, zh)], out.at[pl.ds(sid * zh, zh)])

  @pl.when(core == 0)
  def _():
    flush(h0, o_cs)
    flush(h1, o_cd)

  @pl.when(core == 1)
  def _():
    flush(h0, o_ws)
    flush(h1, o_wd)
    flush(h2, o_bs)
    flush(h3, o_bd)


@functools.cache
def _make_hist():
  deg = jax.ShapeDtypeStruct((NPAD,), F32)
  return pl.kernel(
      _hist_body,
      out_type=(deg,) * 6,
      mesh=plsc.VectorSubcoreMesh(core_axis_name="c", subcore_axis_name="s",
                                  num_cores=NC, num_subcores=NS),
      scratch_types=[
          pltpu.VMEM_SHARED((NPAD,), F32),
          pltpu.VMEM_SHARED((NPAD,), F32),
          pltpu.VMEM_SHARED((NPAD,), F32),
          pltpu.VMEM_SHARED((NPAD,), F32),
          pltpu.VMEM((CRH, G), I32),
          pltpu.VMEM((G,), F32),
          pltpu.VMEM((NPAD // NS,), F32),
      ],
      compiler_params=pltpu.CompilerParams(needs_layout_passes=False),
  )


NBUF = 4


def _segsum_body(slabs, y_c, y_w, y_b, cs, cd, ws, wd, bs, bd, zeros,
                 out_c, out_w, out_b,
                 sbuf_a, dbuf_a, sbuf_b, dbuf_b, rowbufs, accum,
                 gsems, ssems, sem_a, sem_b):
  core = lax.axis_index("c")
  sid = lax.axis_index("s")

  def run_rel(y3d, src2d, dst2d, out, erows):
    trows = erows // NS
    nch = trows // CRH
    pairs = nch // 2
    brow = sid * trows

    def stg(j, sb, db, sem):
      return (pltpu.make_async_copy(src2d.at[pl.ds(brow + j * CRH, CRH)],
                                    sb, sem),
              pltpu.make_async_copy(dst2d.at[pl.ds(brow + j * CRH, CRH)],
                                    db, sem))

    def pipe(s, sb, db):
      def gath(r):
        b = r % NBUF
        return pltpu.make_async_copy(y3d.at[s].at[sb.at[r]],
                                     rowbufs[b], gsems[b])

      def scat(r):
        b = r % NBUF
        return pltpu.make_async_copy(rowbufs[b], accum.at[db.at[r]],
                                     ssems[b])

      gath(0).start()
      gath(1).start()
      for r in range(CRH):
        gath(r).wait()
        pltpu.async_copy(rowbufs[r % NBUF], accum.at[db.at[r]],
                         ssems[r % NBUF], add=True)
        if r + 2 < CRH:
          if r - 2 >= 0:
            scat(r - 2).wait()
          gath(r + 2).start()
      for r in range(max(0, CRH - NBUF), CRH):
        scat(r).wait()

    for s in range(slabs):
      pltpu.sync_copy(zeros, accum.at[pl.ds(sid * FPT, FPT)])
      plsc.subcore_barrier()

      for c in stg(0, sbuf_a, dbuf_a, sem_a):
        c.start()

      def pair(t, _):
        j0 = 2 * t
        for c in stg(j0, sbuf_a, dbuf_a, sem_a):
          c.wait()
        for c in stg(j0 + 1, sbuf_b, dbuf_b, sem_b):
          c.start()
        pipe(s, sbuf_a, dbuf_a)
        for c in stg(j0 + 1, sbuf_b, dbuf_b, sem_b):
          c.wait()

        @pl.when(j0 + 2 < nch)
        def _():
          for c in stg(j0 + 2, sbuf_a, dbuf_a, sem_a):
            c.start()
        pipe(s, sbuf_b, dbuf_b)
        return 0
      lax.fori_loop(0, pairs, pair, 0)

      plsc.subcore_barrier()
      pltpu.sync_copy(accum.at[pl.ds(sid * FPT, FPT)],
                      out.at[s].at[pl.ds(sid * FPT, FPT)])

  @pl.when(core == 0)
  def _():
    run_rel(y_c, cs, cd, out_c, EC_ROWS)

  @pl.when(core == 1)
  def _():
    run_rel(y_w, ws, wd, out_w, EW_ROWS)
    run_rel(y_b, bs, bd, out_b, EW_ROWS)


@functools.cache
def _make_segsum(slabs):
  out = jax.ShapeDtypeStruct((slabs, NPAD, SLAB), F32)
  return pl.kernel(
      functools.partial(_segsum_body, slabs),
      out_type=(out,) * 3,
      mesh=plsc.VectorSubcoreMesh(core_axis_name="c", subcore_axis_name="s",
                                  num_cores=NC, num_subcores=NS),
      scratch_types=[
          pltpu.VMEM((CRH, G), I32),
          pltpu.VMEM((CRH, G), I32),
          pltpu.VMEM((CRH, G), I32),
          pltpu.VMEM((CRH, G), I32),
          [pltpu.VMEM((G, SLAB), F32) for _ in range(NBUF)],
          pltpu.VMEM_SHARED((NPAD, SLAB), F32),
          [pltpu.SemaphoreType.DMA for _ in range(NBUF)],
          [pltpu.SemaphoreType.DMA for _ in range(NBUF)],
          pltpu.SemaphoreType.DMA,
          pltpu.SemaphoreType.DMA,
      ],
      compiler_params=pltpu.CompilerParams(needs_layout_passes=False,
                                           use_tc_tiling_on_sc=False),
  )


def _segsum3(y_c, y_w, y_b, cs, cd, ws, wd, bs, bd, zeros):
  return _make_segsum(y_c.shape[0])(y_c, y_w, y_b, cs, cd, ws, wd, bs, bd,
                                    zeros)


def _rs(d):
  return lax.rsqrt(jnp.maximum(d, 1.0))


def _slabify(o_ref, y):
  w = y.shape[1]
  for s in range(w // SLAB):
    o_ref[s] = y[:, s * SLAB:(s + 1) * SLAB]


def _unslab(p_ref):
  p = p_ref[...]
  return jnp.concatenate([p[s] for s in range(p.shape[0])], axis=1)


def _proj2_body(x_ref, da_ref, db_ref, wa_ref, wb_ref, ya_ref, yb_ref):
  x = x_ref[...]
  _slabify(ya_ref, jnp.dot(x * _rs(da_ref[...]), wa_ref[...],
                           preferred_element_type=F32))
  _slabify(yb_ref, jnp.dot(x * _rs(db_ref[...]), wb_ref[...],
                           preferred_element_type=F32))


def _proj1_body(x_ref, da_ref, wa_ref, ya_ref):
  _slabify(ya_ref, jnp.dot(x_ref[...] * _rs(da_ref[...]), wa_ref[...],
                           preferred_element_type=F32))


def _rowmask():
  i = pl.program_id(0)
  rid = i * MBLK + lax.broadcasted_iota(I32, (MBLK, 1), 0)
  return rid < N_REAL


def _mid2_body(ac_ref, aw_ref, dcd_ref, dwd_ref, b1c_ref, b1w_ref,
               dcs_ref, dbs_ref, w2c_ref, w2b_ref, yc_ref, yb_ref):
  ac = _unslab(ac_ref)
  aw = _unslab(aw_ref)
  h = jax.nn.relu(ac * _rs(dcd_ref[...]) + b1c_ref[...]
                  + aw * _rs(dwd_ref[...]) + b1w_ref[...])
  h = jnp.where(_rowmask(), h, 0.0)
  _slabify(yc_ref, jnp.dot(h * _rs(dcs_ref[...]), w2c_ref[...],
                           preferred_element_type=F32))
  _slabify(yb_ref, jnp.dot(h * _rs(dbs_ref[...]), w2b_ref[...],
                           preferred_element_type=F32))


def _mid1_body(ab_ref, dbd_ref, b1b_ref, dws_ref, w2w_ref, yw_ref):
  ab = _unslab(ab_ref)
  h = jax.nn.relu(ab * _rs(dbd_ref[...]) + b1b_ref[...])
  h = jnp.where(_rowmask(), h, 0.0)
  _slabify(yw_ref, jnp.dot(h * _rs(dws_ref[...]), w2w_ref[...],
                           preferred_element_type=F32))


def _fin2_body(a_ref, b_ref, da_ref, db_ref, ba_ref, bb_ref, o_ref):
  o_ref[...] = (_unslab(a_ref) * _rs(da_ref[...]) + ba_ref[...]
                + _unslab(b_ref) * _rs(db_ref[...]) + bb_ref[...])


def _fin1_body(a_ref, da_ref, ba_ref, o_ref):
  o_ref[...] = _unslab(a_ref) * _rs(da_ref[...]) + ba_ref[...]


def _bs_rows(w):
  return pl.BlockSpec((MBLK, w), lambda i: (i, 0))


def _bs_col():
  return pl.BlockSpec((MBLK, 1), lambda i: (i, 0))


def _bs_full(r, c):
  return pl.BlockSpec((r, c), lambda i: (0, 0))


def _bs_slab(w):
  return pl.BlockSpec((w // SLAB, MBLK, SLAB), lambda i: (0, i, 0))


def _bs_part(w):
  return pl.BlockSpec((w // SLAB, MBLK, SLAB), lambda i: (0, i, 0))


def _y_shape(w):
  return jax.ShapeDtypeStruct((w // SLAB, NPAD, SLAB), F32)


@functools.cache
def _make_proj2():
  return pl.pallas_call(
      _proj2_body,
      grid=(NBLK,),
      in_specs=[_bs_rows(D), _bs_col(), _bs_col(),
                _bs_full(D, H), _bs_full(D, H)],
      out_specs=[_bs_slab(H), _bs_slab(H)],
      out_shape=[_y_shape(H)] * 2,
  )


@functools.cache
def _make_proj1():
  return pl.pallas_call(
      _proj1_body,
      grid=(NBLK,),
      in_specs=[_bs_rows(D), _bs_col(), _bs_full(D, H)],
      out_specs=_bs_slab(H),
      out_shape=_y_shape(H),
  )


@functools.cache
def _make_mid2():
  return pl.pallas_call(
      _mid2_body,
      grid=(NBLK,),
      in_specs=[_bs_part(H), _bs_part(H), _bs_col(), _bs_col(),
                _bs_full(1, H), _bs_full(1, H), _bs_col(), _bs_col(),
                _bs_full(H, O), _bs_full(H, O)],
      out_specs=[_bs_slab(O), _bs_slab(O)],
      out_shape=[_y_shape(O)] * 2,
  )


@functools.cache
def _make_mid1():
  return pl.pallas_call(
      _mid1_body,
      grid=(NBLK,),
      in_specs=[_bs_part(H), _bs_col(), _bs_full(1, H), _bs_col(),
                _bs_full(H, O)],
      out_specs=_bs_slab(O),
      out_shape=_y_shape(O),
  )


@functools.cache
def _make_fin2():
  return pl.pallas_call(
      _fin2_body,
      grid=(NBLK,),
      in_specs=[_bs_part(O), _bs_part(O), _bs_col(), _bs_col(),
                _bs_full(1, O), _bs_full(1, O)],
      out_specs=_bs_rows(O),
      out_shape=jax.ShapeDtypeStruct((NPAD, O), F32),
  )


@functools.cache
def _make_fin1():
  return pl.pallas_call(
      _fin1_body,
      grid=(NBLK,),
      in_specs=[_bs_part(O), _bs_col(), _bs_full(1, O)],
      out_specs=_bs_rows(O),
      out_shape=jax.ShapeDtypeStruct((NPAD, O), F32),
  )


def _pad_edges(ei, rows):
  n = rows * G
  e = ei.shape[1]
  fill = TRASH + jnp.arange(n - e, dtype=I32) % (NPAD - N_REAL)
  src = jnp.concatenate([ei[0].astype(I32), fill]).reshape(rows, G)
  dst = jnp.concatenate([ei[1].astype(I32), fill]).reshape(rows, G)
  return src, dst


def kernel(x_paper, x_author, ei_cites, ei_writes, ei_written_by,
           W1_cites, b1_cites, W1_writes, b1_writes, W1_wb, b1_wb,
           W2_cites, b2_cites, W2_writes, b2_writes, W2_wb, b2_wb):
  xp = jnp.pad(x_paper, ((0, NPAD - N_REAL), (0, 0)))
  xa = jnp.pad(x_author, ((0, NPAD - N_REAL), (0, 0)))
  cs, cd = _pad_edges(ei_cites, EC_ROWS)
  ws, wd = _pad_edges(ei_writes, EW_ROWS)
  bs, bd = _pad_edges(ei_written_by, EW_ROWS)

  deg_cs, deg_cd, deg_ws, deg_wd, deg_bs, deg_bd = _make_hist()(
      cs, cd, ws, wd, bs, bd)
  d_cs = deg_cs.reshape(NPAD, 1)
  d_cd = deg_cd.reshape(NPAD, 1)
  d_ws = deg_ws.reshape(NPAD, 1)
  d_wd = deg_wd.reshape(NPAD, 1)
  d_bs = deg_bs.reshape(NPAD, 1)
  d_bd = deg_bd.reshape(NPAD, 1)

  yc1, yb1 = _make_proj2()(xp, d_cs, d_bs, W1_cites, W1_wb)
  yw1 = _make_proj1()(xa, d_ws, W1_writes)

  zeros = jnp.zeros((FPT, SLAB), F32)
  ac, aw, ab = _segsum3(yc1, yw1, yb1, cs, cd, ws, wd, bs, bd, zeros)

  yc2, yb2 = _make_mid2()(ac, aw, d_cd, d_wd,
                          b1_cites.reshape(1, H), b1_writes.reshape(1, H),
                          d_cs, d_bs, W2_cites, W2_wb)
  yw2 = _make_mid1()(ab, d_bd, b1_wb.reshape(1, H), d_ws, W2_writes)

  ac2, aw2, ab2 = _segsum3(yc2, yw2, yb2, cs, cd, ws, wd, bs, bd, zeros)

  o_paper = _make_fin2()(ac2, aw2, d_cd, d_wd,
                         b2_cites.reshape(1, O), b2_writes.reshape(1, O))
  o_author = _make_fin1()(ab2, d_bd, b2_wb.reshape(1, O))
  return o_paper[:N_REAL], o_author[:N_REAL]


# speedup vs baseline: 1.0814x; 1.0814x over previous
import functools

import jax
import jax.numpy as jnp
from jax import lax
from jax.experimental import pallas as pl
from jax.experimental.pallas import tpu as pltpu
from jax.experimental.pallas import tpu_sc as plsc

F32 = jnp.float32
I32 = jnp.int32

N_REAL = 50000
NPAD = 51200
D = 128
H = 128
O = 64
NC = 2
NS = 16
G = 128
TRASH = N_REAL
SLAB = 32
ZR = 64
CRH = 8

EC_ROWS = 2560
EW_ROWS = 1280

MBLK = 512
NBLK = NPAD // MBLK
FPT = NPAD // NS


def _hist_body(csrc, cdst, wsrc, wdst, bsrc, bdst,
               o_cs, o_cd, o_ws, o_wd, o_bs, o_bd,
               h0, h1, h2, h3, ibuf, ones, zbuf):
  core = lax.axis_index("c")
  sid = lax.axis_index("s")
  zh = NPAD // NS

  def fill_ones(i, _):
    ones[pl.ds(i * 16, 16)] = jnp.full((16,), 1.0, F32)
    return 0
  lax.fori_loop(0, G // 16, fill_ones, 0)

  def fill_z(i, _):
    zbuf[pl.ds(i * 16, 16)] = jnp.zeros((16,), F32)
    return 0
  lax.fori_loop(0, zh // 16, fill_z, 0)

  for h in (h0, h1, h2, h3):
    pltpu.sync_copy(zbuf, h.at[pl.ds(sid * zh, zh)])
  plsc.subcore_barrier()

  def accum(idx2d, hist, trows):
    brow = sid * trows

    def chunk(j, _):
      pltpu.sync_copy(idx2d.at[pl.ds(brow + j * CRH, CRH)], ibuf)

      def row(r, _):
        pltpu.sync_copy(ones, hist.at[ibuf.at[r]], add=True)
        return 0
      lax.fori_loop(0, CRH, row, 0)
      return 0
    lax.fori_loop(0, trows // CRH, chunk, 0)

  @pl.when(core == 0)
  def _():
    accum(csrc, h0, EC_ROWS // NS)
    accum(cdst, h1, EC_ROWS // NS)

  @pl.when(core == 1)
  def _():
    accum(wsrc, h0, EW_ROWS // NS)
    accum(wdst, h1, EW_ROWS // NS)
    accum(bsrc, h2, EW_ROWS // NS)
    accum(bdst, h3, EW_ROWS // NS)

  plsc.subcore_barrier()

  def flush(hist, out):
    pltpu.sync_copy(hist.at[pl.ds(sid * zh, zh)], out.at[pl.ds(sid * zh, zh)])

  @pl.when(core == 0)
  def _():
    flush(h0, o_cs)
    flush(h1, o_cd)

  @pl.when(core == 1)
  def _():
    flush(h0, o_ws)
    flush(h1, o_wd)
    flush(h2, o_bs)
    flush(h3, o_bd)


@functools.cache
def _make_hist():
  deg = jax.ShapeDtypeStruct((NPAD,), F32)
  return pl.kernel(
      _hist_body,
      out_type=(deg,) * 6,
      mesh=plsc.VectorSubcoreMesh(core_axis_name="c", subcore_axis_name="s",
                                  num_cores=NC, num_subcores=NS),
      scratch_types=[
          pltpu.VMEM_SHARED((NPAD,), F32),
          pltpu.VMEM_SHARED((NPAD,), F32),
          pltpu.VMEM_SHARED((NPAD,), F32),
          pltpu.VMEM_SHARED((NPAD,), F32),
          pltpu.VMEM((CRH, G), I32),
          pltpu.VMEM((G,), F32),
          pltpu.VMEM((NPAD // NS,), F32),
      ],
      compiler_params=pltpu.CompilerParams(needs_layout_passes=False),
  )


NBUF = 4


def _segsum_body(slabs, y_c, y_w, y_b, cs, cd, ws, wd, bs, bd, zeros,
                 out_c, out_w, out_b,
                 sbuf_a, dbuf_a, sbuf_b, dbuf_b, rowbufs, accum,
                 gsems, ssems, sem_a, sem_b):
  core = lax.axis_index("c")
  sid = lax.axis_index("s")

  def run_rel(y3d, src2d, dst2d, out, erows):
    trows = erows // NS
    nch = trows // CRH
    pairs = nch // 2
    brow = sid * trows

    def stg(j, sb, db, sem):
      return (pltpu.make_async_copy(src2d.at[pl.ds(brow + j * CRH, CRH)],
                                    sb, sem),
              pltpu.make_async_copy(dst2d.at[pl.ds(brow + j * CRH, CRH)],
                                    db, sem))

    def pipe(s, sb, db):
      def gath(r):
        b = r % NBUF
        return pltpu.make_async_copy(y3d.at[s].at[sb.at[r]],
                                     rowbufs[b], gsems[b])

      def scat(r):
        b = r % NBUF
        return pltpu.make_async_copy(rowbufs[b], accum.at[db.at[r]],
                                     ssems[b])

      gath(0).start()
      gath(1).start()
      for r in range(CRH):
        gath(r).wait()
        pltpu.async_copy(rowbufs[r % NBUF], accum.at[db.at[r]],
                         ssems[r % NBUF], add=True)
        if r + 2 < CRH:
          if r - 2 >= 0:
            scat(r - 2).wait()
          gath(r + 2).start()
      for r in range(max(0, CRH - NBUF), CRH):
        scat(r).wait()

    for s in range(slabs):
      pltpu.sync_copy(zeros, accum.at[pl.ds(sid * FPT, FPT)])
      plsc.subcore_barrier()

      for c in stg(0, sbuf_a, dbuf_a, sem_a):
        c.start()

      def pair(t, _):
        j0 = 2 * t
        for c in stg(j0, sbuf_a, dbuf_a, sem_a):
          c.wait()
        for c in stg(j0 + 1, sbuf_b, dbuf_b, sem_b):
          c.start()
        pipe(s, sbuf_a, dbuf_a)
        for c in stg(j0 + 1, sbuf_b, dbuf_b, sem_b):
          c.wait()

        @pl.when(j0 + 2 < nch)
        def _():
          for c in stg(j0 + 2, sbuf_a, dbuf_a, sem_a):
            c.start()
        pipe(s, sbuf_b, dbuf_b)
        return 0
      lax.fori_loop(0, pairs, pair, 0)

      plsc.subcore_barrier()
      pltpu.sync_copy(accum.at[pl.ds(sid * FPT, FPT)],
                      out.at[s].at[pl.ds(sid * FPT, FPT)])

  @pl.when(core == 0)
  def _():
    run_rel(y_c, cs, cd, out_c, EC_ROWS)

  @pl.when(core == 1)
  def _():
    run_rel(y_w, ws, wd, out_w, EW_ROWS)
    run_rel(y_b, bs, bd, out_b, EW_ROWS)


@functools.cache
def _make_segsum(slabs):
  out = jax.ShapeDtypeStruct((slabs, NPAD, SLAB), F32)
  return pl.kernel(
      functools.partial(_segsum_body, slabs),
      out_type=(out,) * 3,
      mesh=plsc.VectorSubcoreMesh(core_axis_name="c", subcore_axis_name="s",
                                  num_cores=NC, num_subcores=NS),
      scratch_types=[
          pltpu.VMEM((CRH, G), I32),
          pltpu.VMEM((CRH, G), I32),
          pltpu.VMEM((CRH, G), I32),
          pltpu.VMEM((CRH, G), I32),
          [pltpu.VMEM((G, SLAB), F32) for _ in range(NBUF)],
          pltpu.VMEM_SHARED((NPAD, SLAB), F32),
          [pltpu.SemaphoreType.DMA for _ in range(NBUF)],
          [pltpu.SemaphoreType.DMA for _ in range(NBUF)],
          pltpu.SemaphoreType.DMA,
          pltpu.SemaphoreType.DMA,
      ],
      compiler_params=pltpu.CompilerParams(needs_layout_passes=False,
                                           use_tc_tiling_on_sc=False),
  )


def _segsum3(y_c, y_w, y_b, cs, cd, ws, wd, bs, bd, zeros):
  return _make_segsum(y_c.shape[0])(y_c, y_w, y_b, cs, cd, ws, wd, bs, bd,
                                    zeros)


def _rs(d):
  return lax.rsqrt(jnp.maximum(d, 1.0))


def _slabify(o_ref, y):
  w = y.shape[1]
  for s in range(w // SLAB):
    o_ref[s] = y[:, s * SLAB:(s + 1) * SLAB]


def _unslab(p_ref):
  p = p_ref[...]
  return jnp.concatenate([p[s] for s in range(p.shape[0])], axis=1)


def _proj_body(xp_ref, xa_ref, dcs_ref, dbs_ref, dws_ref,
               w1c_ref, w1b_ref, w1w_ref, yc_ref, yb_ref, yw_ref):
  xp = xp_ref[...]
  _slabify(yc_ref, jnp.dot(xp * _rs(dcs_ref[...]), w1c_ref[...],
                           preferred_element_type=F32))
  _slabify(yb_ref, jnp.dot(xp * _rs(dbs_ref[...]), w1b_ref[...],
                           preferred_element_type=F32))
  _slabify(yw_ref, jnp.dot(xa_ref[...] * _rs(dws_ref[...]), w1w_ref[...],
                           preferred_element_type=F32))


def _rowmask():
  i = pl.program_id(0)
  rid = i * MBLK + lax.broadcasted_iota(I32, (MBLK, 1), 0)
  return rid < N_REAL


def _mid_body(ac_ref, aw_ref, ab_ref, dcd_ref, dwd_ref, dbd_ref,
              b1c_ref, b1w_ref, b1b_ref, dcs_ref, dbs_ref, dws_ref,
              w2c_ref, w2b_ref, w2w_ref, yc_ref, yb_ref, yw_ref):
  mask = _rowmask()
  hp = jax.nn.relu(_unslab(ac_ref) * _rs(dcd_ref[...]) + b1c_ref[...]
                   + _unslab(aw_ref) * _rs(dwd_ref[...]) + b1w_ref[...])
  hp = jnp.where(mask, hp, 0.0)
  ha = jax.nn.relu(_unslab(ab_ref) * _rs(dbd_ref[...]) + b1b_ref[...])
  ha = jnp.where(mask, ha, 0.0)
  _slabify(yc_ref, jnp.dot(hp * _rs(dcs_ref[...]), w2c_ref[...],
                           preferred_element_type=F32))
  _slabify(yb_ref, jnp.dot(hp * _rs(dbs_ref[...]), w2b_ref[...],
                           preferred_element_type=F32))
  _slabify(yw_ref, jnp.dot(ha * _rs(dws_ref[...]), w2w_ref[...],
                           preferred_element_type=F32))


def _fin_body(ac_ref, aw_ref, ab_ref, dcd_ref, dwd_ref, dbd_ref,
              b2c_ref, b2w_ref, b2b_ref, op_ref, oa_ref):
  op_ref[...] = (_unslab(ac_ref) * _rs(dcd_ref[...]) + b2c_ref[...]
                 + _unslab(aw_ref) * _rs(dwd_ref[...]) + b2w_ref[...])
  oa_ref[...] = _unslab(ab_ref) * _rs(dbd_ref[...]) + b2b_ref[...]


def _bs_rows(w):
  return pl.BlockSpec((MBLK, w), lambda i: (i, 0))


def _bs_col():
  return pl.BlockSpec((MBLK, 1), lambda i: (i, 0))


def _bs_full(r, c):
  return pl.BlockSpec((r, c), lambda i: (0, 0))


def _bs_slab(w):
  return pl.BlockSpec((w // SLAB, MBLK, SLAB), lambda i: (0, i, 0))


def _y_shape(w):
  return jax.ShapeDtypeStruct((w // SLAB, NPAD, SLAB), F32)


@functools.cache
def _make_proj():
  return pl.pallas_call(
      _proj_body,
      grid=(NBLK,),
      in_specs=[_bs_rows(D), _bs_rows(D), _bs_col(), _bs_col(), _bs_col(),
                _bs_full(D, H), _bs_full(D, H), _bs_full(D, H)],
      out_specs=[_bs_slab(H)] * 3,
      out_shape=[_y_shape(H)] * 3,
  )


@functools.cache
def _make_mid():
  return pl.pallas_call(
      _mid_body,
      grid=(NBLK,),
      in_specs=[_bs_slab(H), _bs_slab(H), _bs_slab(H),
                _bs_col(), _bs_col(), _bs_col(),
                _bs_full(1, H), _bs_full(1, H), _bs_full(1, H),
                _bs_col(), _bs_col(), _bs_col(),
                _bs_full(H, O), _bs_full(H, O), _bs_full(H, O)],
      out_specs=[_bs_slab(O)] * 3,
      out_shape=[_y_shape(O)] * 3,
  )


@functools.cache
def _make_fin():
  return pl.pallas_call(
      _fin_body,
      grid=(NBLK,),
      in_specs=[_bs_slab(O), _bs_slab(O), _bs_slab(O),
                _bs_col(), _bs_col(), _bs_col(),
                _bs_full(1, O), _bs_full(1, O), _bs_full(1, O)],
      out_specs=[_bs_rows(O), _bs_rows(O)],
      out_shape=[jax.ShapeDtypeStruct((NPAD, O), F32)] * 2,
  )


def _pad_edges(ei, rows):
  n = rows * G
  e = ei.shape[1]
  fill = TRASH + jnp.arange(n - e, dtype=I32) % (NPAD - N_REAL)
  src = jnp.concatenate([ei[0].astype(I32), fill]).reshape(rows, G)
  dst = jnp.concatenate([ei[1].astype(I32), fill]).reshape(rows, G)
  return src, dst


def kernel(x_paper, x_author, ei_cites, ei_writes, ei_written_by,
           W1_cites, b1_cites, W1_writes, b1_writes, W1_wb, b1_wb,
           W2_cites, b2_cites, W2_writes, b2_writes, W2_wb, b2_wb):
  xp = jnp.pad(x_paper, ((0, NPAD - N_REAL), (0, 0)))
  xa = jnp.pad(x_author, ((0, NPAD - N_REAL), (0, 0)))
  cs, cd = _pad_edges(ei_cites, EC_ROWS)
  ws, wd = _pad_edges(ei_writes, EW_ROWS)
  bs, bd = _pad_edges(ei_written_by, EW_ROWS)

  deg_cs, deg_cd, deg_ws, deg_wd, deg_bs, deg_bd = _make_hist()(
      cs, cd, ws, wd, bs, bd)
  d_cs = deg_cs.reshape(NPAD, 1)
  d_cd = deg_cd.reshape(NPAD, 1)
  d_ws = deg_ws.reshape(NPAD, 1)
  d_wd = deg_wd.reshape(NPAD, 1)
  d_bs = deg_bs.reshape(NPAD, 1)
  d_bd = deg_bd.reshape(NPAD, 1)

  yc1, yb1, yw1 = _make_proj()(xp, xa, d_cs, d_bs, d_ws,
                               W1_cites, W1_wb, W1_writes)

  zeros = jnp.zeros((FPT, SLAB), F32)
  ac, aw, ab = _segsum3(yc1, yw1, yb1, cs, cd, ws, wd, bs, bd, zeros)

  yc2, yb2, yw2 = _make_mid()(
      ac, aw, ab, d_cd, d_wd, d_bd,
      b1_cites.reshape(1, H), b1_writes.reshape(1, H), b1_wb.reshape(1, H),
      d_cs, d_bs, d_ws, W2_cites, W2_wb, W2_writes)

  ac2, aw2, ab2 = _segsum3(yc2, yw2, yb2, cs, cd, ws, wd, bs, bd, zeros)

  o_paper, o_author = _make_fin()(
      ac2, aw2, ab2, d_cd, d_wd, d_bd,
      b2_cites.reshape(1, O), b2_writes.reshape(1, O), b2_wb.reshape(1, O))
  return o_paper[:N_REAL], o_author[:N_REAL]
